# Initial kernel scaffold; baseline (speedup 1.0000x reference)
import jax
import jax.numpy as jnp
from jax.experimental import pallas as pl


def kernel(probs, x_reg):
    raise NotImplementedError("write your pallas kernel here")


# speedup vs baseline: 30.9552x; 30.9552x over previous
import numpy as np
import jax
import jax.numpy as jnp
from jax import lax
from jax.experimental import pallas as pl
from jax.experimental.pallas import tpu as pltpu

_H_FEAT = 50
_W_FEAT = 50
_STRIDE = 16
_SCALES = (8.0, 16.0, 32.0)
_RATIOS = (0.5, 1.0, 2.0)
_SIZE_BASE = 16.0
_IMG_H = 800.0
_IMG_W = 800.0
_PRE_NMS = 6000
_POST_NMS = 1000
_NMS_THRESH = 0.7
_NUM_ANCHORS = _H_FEAT * _W_FEAT * 9

_N_SORT = 32768
_R = 256
_C = 128
_RK = 48
_BIG = 0x7FFFFFF0


def _cell_anchors():
    cell = []
    for r in _RATIOS:
        for s in _SCALES:
            w = _SIZE_BASE * s * np.sqrt(1.0 / r)
            h = _SIZE_BASE * s * np.sqrt(r)
            cell.append([-0.5 * (w - 1.0), -0.5 * (h - 1.0),
                         0.5 * (w - 1.0), 0.5 * (h - 1.0)])
    return np.asarray(cell, dtype=np.float32)


_CELL = _cell_anchors()


def _pick9(a9, consts):
    out = jnp.full(a9.shape, float(consts[8]), dtype=jnp.float32)
    for t in range(8):
        out = jnp.where(a9 == t, jnp.float32(float(consts[t])), out)
    return out


def _nms_sort_body(fg_ref, dx_ref, dy_ref, dw_ref, dh_ref, out_ref):
    f32 = jnp.float32
    row_i = lax.broadcasted_iota(jnp.int32, (_R, _C), 0)
    col_i = lax.broadcasted_iota(jnp.int32, (_R, _C), 1)
    lin = row_i * _C + col_i

    fg = fg_ref[...]
    valid = lin < _NUM_ANCHORS
    key = jnp.where(valid, lax.bitcast_convert_type(fg, jnp.int32), jnp.int32(-1))
    idx = lin

    a9 = lin % 9
    cell = lin // 9
    gx = cell % _W_FEAT
    gy = cell // _W_FEAT
    sx = (gx * _STRIDE).astype(f32)
    sy = (gy * _STRIDE).astype(f32)
    x1a = sx + _pick9(a9, _CELL[:, 0])
    y1a = sy + _pick9(a9, _CELL[:, 1])
    x2a = sx + _pick9(a9, _CELL[:, 2])
    y2a = sy + _pick9(a9, _CELL[:, 3])

    widths = x2a - x1a + 1.0
    heights = y2a - y1a + 1.0
    ctr_x = x1a + 0.5 * widths
    ctr_y = y1a + 0.5 * heights
    dx = dx_ref[...]
    dy = dy_ref[...]
    dw = dw_ref[...]
    dh = dh_ref[...]
    pcx = dx * widths + ctr_x
    pcy = dy * heights + ctr_y
    pw = jnp.exp(dw) * widths
    ph = jnp.exp(dh) * heights
    bx1 = jnp.clip(pcx - 0.5 * pw, 0.0, _IMG_W - 1.0)
    by1 = jnp.clip(pcy - 0.5 * ph, 0.0, _IMG_H - 1.0)
    bx2 = jnp.clip(pcx + 0.5 * pw, 0.0, _IMG_W - 1.0)
    by2 = jnp.clip(pcy + 0.5 * ph, 0.0, _IMG_H - 1.0)

    arrays = [key, idx, bx1, by1, bx2, by2]
    k = 2
    while k <= _N_SORT:
        dir_a = (lin & k) == 0
        j = k // 2
        while j >= 1:
            lowm = (lin & j) == 0
            if j < _C:
                ax, sh = 1, j
            else:
                ax, sh = 0, j // _C

            def pr(x):
                return jnp.where(lowm, jnp.roll(x, -sh, axis=ax),
                                 jnp.roll(x, sh, axis=ax))

            partners = [pr(x) for x in arrays]
            pk, pi = partners[0], partners[1]
            kk, ii = arrays[0], arrays[1]
            self_first = (kk > pk) | ((kk == pk) & (ii < pi))
            take_self = self_first == (lowm == dir_a)
            arrays = [jnp.where(take_self, x, p)
                      for x, p in zip(arrays, partners)]
            j //= 2
        k *= 2

    x1s = arrays[2][:_RK]
    y1s = arrays[3][:_RK]
    x2s = arrays[4][:_RK]
    y2s = arrays[5][:_RK]
    lin48 = lin[:_RK]
    areas = (x2s - x1s + 1.0) * (y2s - y1s + 1.0)
    sup0 = (lin48 >= _PRE_NMS).astype(jnp.int32)

    lane = lax.broadcasted_iota(jnp.int32, (1, _C), 1)

    def step(kstep, sup):
        cand = jnp.where(sup != 0, _BIG, lin48)
        sel = jnp.min(cand)
        m = lin48 == sel
        xx1 = jnp.sum(jnp.where(m, x1s, 0.0))
        yy1 = jnp.sum(jnp.where(m, y1s, 0.0))
        xx2 = jnp.sum(jnp.where(m, x2s, 0.0))
        yy2 = jnp.sum(jnp.where(m, y2s, 0.0))
        a_sel = (xx2 - xx1 + 1.0) * (yy2 - yy1 + 1.0)
        iw = jnp.maximum(jnp.minimum(xx2, x2s) - jnp.maximum(xx1, x1s) + 1.0, 0.0)
        ih = jnp.maximum(jnp.minimum(yy2, y2s) - jnp.maximum(yy1, y1s) + 1.0, 0.0)
        inter = iw * ih
        iou = inter / (a_sel + areas - inter)
        sup = sup | (iou > _NMS_THRESH).astype(jnp.int32)
        row = jnp.zeros((1, _C), dtype=f32)
        row = jnp.where(lane == 1, xx1, row)
        row = jnp.where(lane == 2, yy1, row)
        row = jnp.where(lane == 3, xx2, row)
        row = jnp.where(lane == 4, yy2, row)
        out_ref[pl.ds(kstep, 1), :] = row
        return sup

    lax.fori_loop(0, _POST_NMS, step, sup0)


def _run(probs, x_reg, interpret=False):
    f32 = jnp.float32
    fg = probs[0, :, 1]
    fg = jnp.pad(fg, (0, _N_SORT - _NUM_ANCHORS)).reshape(_R, _C)
    d = jnp.pad(x_reg[0], ((0, _N_SORT - _NUM_ANCHORS), (0, 0)))
    dx = d[:, 0].reshape(_R, _C)
    dy = d[:, 1].reshape(_R, _C)
    dw = d[:, 2].reshape(_R, _C)
    dh = d[:, 3].reshape(_R, _C)
    out = pl.pallas_call(
        _nms_sort_body,
        out_shape=jax.ShapeDtypeStruct((1024, _C), f32),
        interpret=interpret,
    )(fg, dx, dy, dw, dh)
    return out[:_POST_NMS, :5].reshape(1, _POST_NMS, 5)


def kernel(probs, x_reg):
    return _run(probs, x_reg)
